# Initial kernel scaffold; baseline (speedup 1.0000x reference)
import jax
import jax.numpy as jnp
from jax.experimental import pallas as pl


def kernel(conv1_w, conv1_b, conv2_w, conv2_b, conv3_w, conv3_b, conv4_w, conv4_b, conv5_w, conv5_b, lin1_w, lin1_b, lin2_w, lin2_b, x):
    raise NotImplementedError("write your pallas kernel here")


# speedup vs baseline: 4.7828x; 4.7828x over previous
import jax
import jax.numpy as jnp
from jax import lax
from jax.experimental import pallas as pl
from jax.experimental.pallas import tpu as pltpu

_B = 16
_NCLS = 5
_S = 96
_BS = _B * _S
_SELK = _BS - 2 * 8 - 8


def _fill_strips(src, lhs_ref, y0, gr, k, c):
    for g in range(gr):
        for di in range(k):
            lhs_ref[di * c:(di + 1) * c, g * _BS:(g + 1) * _BS] = (
                src[y0 + g + di])


def _conv1(x_ref, w_ref, b_ref, lhs_ref, act_ref):
    k, c, o, gr = 5, 8, 64, 8
    kc = k * c
    lw = gr * _BS
    pw = _BS - 1
    bias = jnp.broadcast_to(b_ref[...], (o, pw))
    lhs_ref[:, lw - (k - 1):lw] = jnp.zeros((k * kc, k - 1), jnp.bfloat16)
    for gi in range(80 // gr):
        _fill_strips(x_ref, lhs_ref, gi * gr, gr, k, c)
        for dj in range(1, k):
            lhs_ref[dj * kc:(dj + 1) * kc, 0:lw - dj] = lhs_ref[0:kc, dj:lw]
        z = jnp.dot(w_ref[...], lhs_ref[...],
                    preferred_element_type=jnp.float32)
        for u in range(gr // 2):
            a = z[:, (2 * u) * _BS:(2 * u + 1) * _BS]
            b = z[:, (2 * u + 1) * _BS:(2 * u + 2) * _BS]
            m = jnp.maximum(a, b)
            p = jnp.maximum(m[:, 0:pw], m[:, 1:pw + 1])
            act_ref[gi * (gr // 2) + u, :, 0:pw] = jnp.maximum(
                p + bias, 0.0).astype(jnp.bfloat16)


def _conv_mid(src_ref, w_ref, b_ref, lhs_ref, act_ref, c, o, d, pool, groups):
    k = 3
    mw = _BS - (k - 1) * d
    pw = mw - d if pool else mw
    bias = jnp.broadcast_to(b_ref[...], (o, pw))
    for (y0, gr, p0) in groups:
        _fill_strips(src_ref, lhs_ref, y0, gr, k, c)
        lw = gr * _BS
        nv = lw - (k - 1) * d
        z = jnp.dot(w_ref[...], lhs_ref[:, 0:lw],
                    preferred_element_type=jnp.float32)
        conv = z[0:o, 0:nv]
        for dj in range(1, k):
            conv = conv + z[dj * o:(dj + 1) * o, d * dj:d * dj + nv]
        if pool:
            for u in range(gr // 2):
                a = conv[:, (2 * u) * _BS:(2 * u) * _BS + mw]
                b = conv[:, (2 * u + 1) * _BS:(2 * u + 1) * _BS + mw]
                m = jnp.maximum(a, b)
                p = jnp.maximum(m[:, 0:pw], m[:, d:d + pw])
                act_ref[p0 + u, :, 0:pw] = jnp.maximum(
                    p + bias, 0.0).astype(jnp.bfloat16)
        else:
            for g in range(gr):
                row = conv[:, g * _BS:g * _BS + mw]
                act_ref[p0 + g, :, 0:pw] = jnp.maximum(
                    row + bias, 0.0).astype(jnp.bfloat16)


def _head(act5_ref, sel_ref, w1_ref, b1_ref, w2_ref, b2_ref, feat_ref, o_ref):
    g0 = jnp.dot(act5_ref[0][:, 0:_SELK], sel_ref[...],
                 preferred_element_type=jnp.float32)
    g1 = jnp.dot(act5_ref[1][:, 0:_SELK], sel_ref[...],
                 preferred_element_type=jnp.float32)
    a00, a01 = g0[:, 0:_B], g0[:, _B:2 * _B]
    a10, a11 = g1[:, 0:_B], g1[:, _B:2 * _B]
    lvl1 = jnp.maximum(jnp.maximum(a00, a01), jnp.maximum(a10, a11))
    feat_ref[0:32, :] = lvl1
    feat_ref[32:64, :] = a00
    feat_ref[64:96, :] = a01
    feat_ref[96:128, :] = a10
    feat_ref[128:160, :] = a11
    h = jnp.dot(w1_ref[...], feat_ref[...].astype(jnp.bfloat16),
                preferred_element_type=jnp.float32) + b1_ref[...]
    logits = jnp.dot(w2_ref[...], h.astype(jnp.bfloat16),
                     preferred_element_type=jnp.float32) + b2_ref[...]
    mx = jnp.max(logits, axis=0, keepdims=True)
    e = jnp.exp(logits - mx)
    o_ref[...] = e / jnp.sum(e, axis=0, keepdims=True)


def _body(x_ref, w1c_ref, b1c_ref, w2c_ref, b2c_ref, w3c_ref, b3c_ref,
          w4c_ref, b4c_ref, w5c_ref, b5c_ref, sel_ref, w1_ref, b1_ref,
          w2_ref, b2_ref, o_ref, act1, act2, act3, act4, act5,
          lhs1, lhs2, lhs3, lhs45, feat):
    act1[:, :, _BS - 8:_BS] = jnp.zeros((40, 64, 8), jnp.bfloat16)
    act2[:, :, _BS - 8:_BS] = jnp.zeros((19, 32, 8), jnp.bfloat16)
    act3[:, :, _BS - 16:_BS] = jnp.zeros((8, 32, 16), jnp.bfloat16)
    act4[:, :, _BS - 24:_BS] = jnp.zeros((6, 32, 24), jnp.bfloat16)
    _conv1(x_ref, w1c_ref, b1c_ref, lhs1, act1)
    _conv_mid(act1, w2c_ref, b2c_ref, lhs2, act2, 64, 32, 2, True,
              [(0, 8, 0), (8, 8, 4), (16, 8, 8), (24, 8, 12), (32, 6, 16)])
    _conv_mid(act2, w3c_ref, b3c_ref, lhs3, act3, 32, 32, 4, True,
              [(0, 8, 0), (8, 8, 4)])
    _conv_mid(act3, w4c_ref, b4c_ref, lhs45, act4, 32, 32, 8, False,
              [(0, 6, 0)])
    _conv_mid(act4, w5c_ref, b5c_ref, lhs45, act5, 32, 32, 8, True,
              [(0, 4, 0)])
    _head(act5, sel_ref, w1_ref, b1_ref, w2_ref, b2_ref, feat, o_ref)


def kernel(conv1_w, conv1_b, conv2_w, conv2_b, conv3_w, conv3_b,
           conv4_w, conv4_b, conv5_w, conv5_b,
           lin1_w, lin1_b, lin2_w, lin2_b, x):
    n = x.shape[0]
    nb = n // _B

    y = x.reshape(nb, _B, 3, 84, 84)
    y = jnp.pad(y, ((0, 0), (0, 0), (0, 5), (0, 0), (0, _S - 84)))
    y = jnp.transpose(y, (0, 3, 2, 1, 4)).reshape(nb, 84, 8, _BS)
    y = y.astype(jnp.bfloat16)

    w1c = jnp.pad(conv1_w, ((0, 0), (0, 5), (0, 0), (0, 0)))
    w1c = jnp.transpose(w1c, (0, 3, 2, 1)).reshape(64, 200).astype(jnp.bfloat16)
    def wstack(w, k, c, o):
        return jnp.transpose(w, (3, 0, 2, 1)).reshape(k * o, k * c).astype(
            jnp.bfloat16)
    w2c = wstack(conv2_w, 3, 64, 32)
    w3c = wstack(conv3_w, 3, 32, 32)
    w4c = wstack(conv4_w, 3, 32, 32)
    w5c = wstack(conv5_w, 3, 32, 32)
    bias = lambda b: b.reshape(-1, 1).astype(jnp.float32)
    sel = jnp.zeros((_SELK, 2 * _B), jnp.bfloat16)
    for b in range(_B):
        sel = sel.at[b * _S, b].set(1).at[b * _S + 16, _B + b].set(1)
    c = 32
    perm = list(range(c)) + [c + ch * 4 + g for g in range(4) for ch in range(c)]
    w1p = lin1_w[:, jnp.array(perm, dtype=jnp.int32)].astype(jnp.bfloat16)

    operands = [
        y,
        w1c, bias(conv1_b), w2c, bias(conv2_b), w3c, bias(conv3_b),
        w4c, bias(conv4_b), w5c, bias(conv5_b),
        sel, w1p, bias(lin1_b), lin2_w.astype(jnp.bfloat16), bias(lin2_b),
    ]
    in_specs = [pl.BlockSpec((None, 84, 8, _BS), lambda b: (b, 0, 0, 0))]
    in_specs += [pl.BlockSpec(op.shape, lambda b: (0,) * op.ndim)
                 for op in operands[1:]]

    scratch_shapes = [
        pltpu.VMEM((40, 64, _BS), jnp.bfloat16),
        pltpu.VMEM((19, 32, _BS), jnp.bfloat16),
        pltpu.VMEM((8, 32, _BS), jnp.bfloat16),
        pltpu.VMEM((6, 32, _BS), jnp.bfloat16),
        pltpu.VMEM((2, 32, _BS), jnp.bfloat16),
        pltpu.VMEM((200, 8 * _BS), jnp.bfloat16),
        pltpu.VMEM((192, 8 * _BS), jnp.bfloat16),
        pltpu.VMEM((96, 8 * _BS), jnp.bfloat16),
        pltpu.VMEM((96, 6 * _BS), jnp.bfloat16),
        pltpu.VMEM((160, _B), jnp.float32),
    ]

    out = pl.pallas_call(
        _body,
        out_shape=jax.ShapeDtypeStruct((nb, _NCLS, _B), jnp.float32),
        grid=(nb,),
        in_specs=in_specs,
        out_specs=pl.BlockSpec((None, _NCLS, _B), lambda b: (b, 0, 0)),
        scratch_shapes=scratch_shapes,
        compiler_params=pltpu.CompilerParams(
            dimension_semantics=("parallel",)),
    )(*operands)
    return jnp.transpose(out, (0, 2, 1)).reshape(n, _NCLS)
